# baseline (speedup 1.0000x reference)
import functools

import jax
import jax.numpy as jnp
from jax import lax
from jax.experimental import pallas as pl
from jax.experimental.pallas import tpu as pltpu
from jax.experimental.pallas import tpu_sc as plsc

NUM_CODES = 1024
CODE_DIM = 64
ROWS_PER_BLOCK = 512


def _codes_body(x_ref, rn_ref, cbm2_ref, cn_ref, iota_ref, out_ref):
    x = x_ref[...]
    cbm2 = cbm2_ref[...]
    m2 = lax.dot_general(x, cbm2, (((1,), (1,)), ((), ())),
                         preferred_element_type=jnp.float32)
    d2 = (rn_ref[...] + m2) + cn_ref[...]
    mn = jnp.min(d2, axis=1, keepdims=True)
    big = jnp.where(d2 == mn, iota_ref[...], float(NUM_CODES))
    out_ref[0, 0, :] = jnp.min(big, axis=1).astype(jnp.int32)


def _compute_codes(flat, rn, cbm2, cn, iota_f):
    n = flat.shape[0]
    nblk = n // ROWS_PER_BLOCK
    codes3 = pl.pallas_call(
        _codes_body,
        grid=(nblk,),
        in_specs=[
            pl.BlockSpec((ROWS_PER_BLOCK, CODE_DIM), lambda i: (i, 0)),
            pl.BlockSpec((ROWS_PER_BLOCK, 1), lambda i: (i, 0)),
            pl.BlockSpec((NUM_CODES, CODE_DIM), lambda i: (0, 0)),
            pl.BlockSpec((1, NUM_CODES), lambda i: (0, 0)),
            pl.BlockSpec((1, NUM_CODES), lambda i: (0, 0)),
        ],
        out_specs=pl.BlockSpec((1, 1, ROWS_PER_BLOCK), lambda i: (i, 0, 0)),
        out_shape=jax.ShapeDtypeStruct((nblk, 1, ROWS_PER_BLOCK), jnp.int32),
    )(flat, rn, cbm2, cn, iota_f)
    return codes3.reshape(n)


def _make_sc_gather(n_rows):
    info = plsc.get_sparse_core_info()
    nw = info.num_cores * info.num_subcores
    b_per_w = n_rows // nw
    mesh = plsc.VectorSubcoreMesh(core_axis_name="c", subcore_axis_name="s")

    @functools.partial(
        pl.kernel,
        mesh=mesh,
        out_type=jax.ShapeDtypeStruct((n_rows, CODE_DIM), jnp.float32),
        scratch_types=[
            pltpu.VMEM((b_per_w,), jnp.int32),
            pltpu.VMEM((b_per_w, CODE_DIM), jnp.float32),
            pltpu.SemaphoreType.DMA,
        ],
        compiler_params=pltpu.CompilerParams(use_tc_tiling_on_sc=False),
    )
    def gather(table_hbm, idx_hbm, out_hbm, idx_v, rows_v, sem):
        wid = lax.axis_index("s") * info.num_cores + lax.axis_index("c")
        base = wid * b_per_w
        pltpu.sync_copy(idx_hbm.at[pl.ds(base, b_per_w)], idx_v)
        pltpu.async_copy(table_hbm.at[idx_v], rows_v, sem).wait()
        pltpu.sync_copy(rows_v, out_hbm.at[pl.ds(base, b_per_w)])

    return gather


def kernel(inputs, codebook):
    b, s, d = inputs.shape
    n = b * s
    flat = inputs.reshape(n, d)
    cbm2 = -2.0 * codebook
    cn = jnp.sum(codebook * codebook, axis=1)[None, :]
    rn = jnp.sum(flat * flat, axis=1, keepdims=True)
    iota_f = jnp.arange(NUM_CODES, dtype=jnp.float32)[None, :]
    codes_flat = _compute_codes(flat, rn, cbm2, cn, iota_f)
    quantized = _make_sc_gather(n)(codebook, codes_flat)
    return quantized.reshape(inputs.shape), codes_flat.reshape(b, s)


# speedup vs baseline: 1.0197x; 1.0197x over previous
import functools

import jax
import jax.numpy as jnp
from jax import lax
from jax.experimental import pallas as pl
from jax.experimental.pallas import tpu as pltpu
from jax.experimental.pallas import tpu_sc as plsc

NUM_CODES = 1024
CODE_DIM = 64
ROWS_PER_BLOCK = 512


def _codes_body(x_ref, rn_ref, cbm2_ref, cn_ref, iota_ref, out_ref):
    x = x_ref[...]
    cbm2 = cbm2_ref[...]
    m2 = lax.dot_general(x, cbm2, (((1,), (1,)), ((), ())),
                         preferred_element_type=jnp.float32)
    rn = rn_ref[...]
    cn = cn_ref[...]
    best = None
    bcid = None
    for c in range(NUM_CODES // 128):
        sl = slice(c * 128, (c + 1) * 128)
        d2c = (rn + m2[:, sl]) + cn[:, sl]
        if best is None:
            best = d2c
            bcid = jnp.zeros(d2c.shape, jnp.float32)
        else:
            upd = d2c < best
            best = jnp.where(upd, d2c, best)
            bcid = jnp.where(upd, float(c), bcid)
    fidx = bcid * 128.0 + iota_ref[...]
    mn = jnp.min(best, axis=1, keepdims=True)
    big = jnp.where(best == mn, fidx, float(NUM_CODES))
    out_ref[0, 0, :] = jnp.min(big, axis=1).astype(jnp.int32)


def _compute_codes(flat, rn, cbm2, cn, iota_f):
    n = flat.shape[0]
    nblk = n // ROWS_PER_BLOCK
    codes3 = pl.pallas_call(
        _codes_body,
        grid=(nblk,),
        in_specs=[
            pl.BlockSpec((ROWS_PER_BLOCK, CODE_DIM), lambda i: (i, 0)),
            pl.BlockSpec((ROWS_PER_BLOCK, 1), lambda i: (i, 0)),
            pl.BlockSpec((NUM_CODES, CODE_DIM), lambda i: (0, 0)),
            pl.BlockSpec((1, NUM_CODES), lambda i: (0, 0)),
            pl.BlockSpec((1, 128), lambda i: (0, 0)),
        ],
        out_specs=pl.BlockSpec((1, 1, ROWS_PER_BLOCK), lambda i: (i, 0, 0)),
        out_shape=jax.ShapeDtypeStruct((nblk, 1, ROWS_PER_BLOCK), jnp.int32),
    )(flat, rn, cbm2, cn, iota_f)
    return codes3.reshape(n)


def _make_sc_gather(n_rows):
    info = plsc.get_sparse_core_info()
    nw = info.num_cores * info.num_subcores
    b_per_w = n_rows // nw
    mesh = plsc.VectorSubcoreMesh(core_axis_name="c", subcore_axis_name="s")

    @functools.partial(
        pl.kernel,
        mesh=mesh,
        out_type=jax.ShapeDtypeStruct((n_rows, CODE_DIM), jnp.float32),
        scratch_types=[
            pltpu.VMEM((b_per_w,), jnp.int32),
            pltpu.VMEM((b_per_w, CODE_DIM), jnp.float32),
            pltpu.SemaphoreType.DMA,
        ],
        compiler_params=pltpu.CompilerParams(use_tc_tiling_on_sc=False),
    )
    def gather(table_hbm, idx_hbm, out_hbm, idx_v, rows_v, sem):
        wid = lax.axis_index("s") * info.num_cores + lax.axis_index("c")
        base = wid * b_per_w
        pltpu.sync_copy(idx_hbm.at[pl.ds(base, b_per_w)], idx_v)
        pltpu.async_copy(table_hbm.at[idx_v], rows_v, sem).wait()
        pltpu.sync_copy(rows_v, out_hbm.at[pl.ds(base, b_per_w)])

    return gather


def kernel(inputs, codebook):
    b, s, d = inputs.shape
    n = b * s
    flat = inputs.reshape(n, d)
    cbm2 = -2.0 * codebook
    cn = jnp.sum(codebook * codebook, axis=1)[None, :]
    rn = jnp.sum(flat * flat, axis=1, keepdims=True)
    iota_f = jnp.arange(128, dtype=jnp.float32)[None, :]
    codes_flat = _compute_codes(flat, rn, cbm2, cn, iota_f)
    quantized = _make_sc_gather(n)(codebook, codes_flat)
    return quantized.reshape(inputs.shape), codes_flat.reshape(b, s)
